# Initial kernel scaffold; baseline (speedup 1.0000x reference)
import jax
import jax.numpy as jnp
from jax.experimental import pallas as pl


def kernel(cxt_feats, cxt_lfeats, obj_attn, wo_obj_idx, dist, fc7_norm_w, lfeat_norm_w, fc_w, fc_b):
    raise NotImplementedError("write your pallas kernel here")


# speedup vs baseline: 3.3544x; 3.3544x over previous
import functools

import jax
import jax.numpy as jnp
from jax.experimental import pallas as pl

SENT = 64
ANN = 256
FC7 = 2048
JEMB = 512
NCH = 6

HI = jax.lax.Precision.HIGHEST


def _fuse_kernel(attn_blk, attn_full, cxt_feats, cw, w1s, w2s, b2,
                 fuse, dists, maxid):
    sb = attn_blk.shape[0]

    attn = attn_blk[...]
    m = jnp.max(attn, axis=1, keepdims=True)
    cols = jax.lax.broadcasted_iota(jnp.int32, (sb, ANN), 1)
    loc_id = jnp.min(jnp.where(attn == m, cols, ANN), axis=1,
                     keepdims=True)
    onehot = (cols == loc_id).astype(jnp.float32)
    mask = (m == 0.0)

    rf = jax.lax.dot(onehot, cxt_feats[...], precision=HI)
    n = jnp.sqrt(jnp.sum(rf * rf, axis=1, keepdims=True))
    inv7 = jnp.where(mask, 0.0, 1.0 / jnp.maximum(n, 1e-12))
    base = jax.lax.dot_general(rf * inv7, w1s[...],
                               (((1,), (1,)), ((), ())),
                               precision=HI)
    base = base + b2[...]

    g = jax.lax.dot(onehot, cw[...], precision=HI)
    lf = [g[:, c * ANN:(c + 1) * ANN] for c in range(5)]
    dg = g[:, 5 * ANN:6 * ANN]

    ss = lf[0] * lf[0]
    for c in range(1, 5):
        ss = ss + lf[c] * lf[c]
    invl = 1.0 / jnp.maximum(jnp.sqrt(ss), 1e-12)
    invl = jnp.where(mask, 0.0, invl)

    w2 = w2s[...]
    out = jnp.broadcast_to(base[:, None, :], (sb, ANN, JEMB))
    for c in range(5):
        out = out + (lf[c] * invl)[:, :, None] * w2[c][None, None, :]
    fuse[...] = out

    dists[...] = jnp.where(mask, 100.0, dg)

    af = attn_full[...]
    mf = jnp.max(af, axis=1, keepdims=True)
    colsf = jax.lax.broadcasted_iota(jnp.int32, (SENT, ANN), 1)
    maxid[...] = jnp.min(jnp.where(af == mf, colsf, ANN), axis=1,
                         keepdims=True)


@functools.partial(jax.jit, static_argnames=("interpret",))
def _run(cxt_feats, cxt_lfeats, obj_attn, dist, fc7_norm_w, lfeat_norm_w,
         fc_w, fc_b, interpret=False):
    lt = jnp.transpose(cxt_lfeats, (1, 2, 0))
    dt = jnp.transpose(dist, (1, 2, 0))
    cw = jnp.concatenate([lt, dt], axis=1).reshape(ANN, NCH * ANN)
    w1s = fc_w[:, :FC7] * fc7_norm_w
    w2s = fc_w[:, FC7:].T * lfeat_norm_w.T
    b2 = fc_b.reshape(1, JEMB)

    sb = 8
    grid = (SENT // sb,)
    fuse, dists, maxid = pl.pallas_call(
        _fuse_kernel,
        grid=grid,
        in_specs=[
            pl.BlockSpec((sb, ANN), lambda i: (i, 0)),
            pl.BlockSpec((SENT, ANN), lambda i: (0, 0)),
            pl.BlockSpec((ANN, FC7), lambda i: (0, 0)),
            pl.BlockSpec((ANN, NCH * ANN), lambda i: (0, 0)),
            pl.BlockSpec((JEMB, FC7), lambda i: (0, 0)),
            pl.BlockSpec((5, JEMB), lambda i: (0, 0)),
            pl.BlockSpec((1, JEMB), lambda i: (0, 0)),
        ],
        out_specs=[
            pl.BlockSpec((sb, ANN, JEMB), lambda i: (i, 0, 0)),
            pl.BlockSpec((sb, ANN), lambda i: (i, 0)),
            pl.BlockSpec((SENT, 1), lambda i: (0, 0)),
        ],
        out_shape=[
            jax.ShapeDtypeStruct((SENT, ANN, JEMB), jnp.float32),
            jax.ShapeDtypeStruct((SENT, ANN), jnp.float32),
            jax.ShapeDtypeStruct((SENT, 1), jnp.int32),
        ],
        interpret=interpret,
    )(obj_attn, obj_attn, cxt_feats, cw, w1s, w2s, b2)
    return fuse, dists, maxid[:, 0]


def kernel(cxt_feats, cxt_lfeats, obj_attn, wo_obj_idx, dist,
           fc7_norm_w, lfeat_norm_w, fc_w, fc_b):
    del wo_obj_idx
    return _run(cxt_feats, cxt_lfeats, obj_attn, dist, fc7_norm_w,
                lfeat_norm_w, fc_w, fc_b)


# speedup vs baseline: 5.4623x; 1.6284x over previous
import functools

import jax
import jax.numpy as jnp
from jax.experimental import pallas as pl
from jax.experimental.pallas import tpu as pltpu

SENT = 64
ANN = 256
FC7 = 2048
JEMB = 512

HI = jax.lax.Precision.HIGHEST


def _prologue_kernel(attn, cxt_feats, dist_t, w1s, b2,
                     maxid, dists, base, okf):
    a = attn[...]
    m = jnp.max(a, axis=1, keepdims=True)
    cols = jax.lax.broadcasted_iota(jnp.int32, (SENT, ANN), 1)
    ids = jnp.min(jnp.where(a == m, cols, ANN), axis=1,
                  keepdims=True)
    maxid[...] = ids
    onehot = (cols == ids).astype(jnp.float32)
    ok = jnp.where(m == 0.0, 0.0, 1.0)
    okf[...] = ok

    dg = jax.lax.dot(onehot, dist_t[...], precision=HI)
    dists[...] = jnp.where(ok == 0.0, 100.0, dg)

    rf = jax.lax.dot(onehot, cxt_feats[...], precision=HI)
    n = jnp.sqrt(jnp.sum(rf * rf, axis=1, keepdims=True))
    inv7 = ok / jnp.maximum(n, 1e-12)
    base[...] = jax.lax.dot_general(rf * inv7, w1s[...],
                                    (((1,), (1,)), ((), ())),
                                    precision=HI) + b2[...]


def _fuse_kernel(ids_ref, cw, base, okf, w2s, fuse, scratch):
    sb = base.shape[0]
    pid = pl.program_id(0)

    for k in range(sb):
        idx = ids_ref[pid * sb + k]
        scratch[pl.ds(k, 1), :] = cw[pl.ds(idx, 1), :]
    g = scratch[...]

    lf = [g[:, c * ANN:(c + 1) * ANN] for c in range(5)]
    ss = lf[0] * lf[0]
    for c in range(1, 5):
        ss = ss + lf[c] * lf[c]
    invl = okf[...] / jnp.maximum(jnp.sqrt(ss), 1e-12)

    w2 = w2s[...]
    out = jnp.broadcast_to(base[...][:, None, :], (sb, ANN, JEMB))
    for c in range(5):
        out = out + (lf[c] * invl)[:, :, None] * w2[c][None, None, :]
    fuse[...] = out


@functools.partial(jax.jit, static_argnames=("interpret",))
def _run(cxt_feats, cxt_lfeats, obj_attn, dist, fc7_norm_w, lfeat_norm_w,
         fc_w, fc_b, interpret=False):
    cw = jnp.transpose(cxt_lfeats, (1, 2, 0)).reshape(ANN, 5 * ANN)
    dist_t = jnp.transpose(dist[:, :, 0], (1, 0))
    w1s = fc_w[:, :FC7] * fc7_norm_w
    w2s = fc_w[:, FC7:].T * lfeat_norm_w.T
    b2 = fc_b.reshape(1, JEMB)

    maxid, dists, base, okf = pl.pallas_call(
        _prologue_kernel,
        in_specs=[
            pl.BlockSpec((SENT, ANN), lambda: (0, 0)),
            pl.BlockSpec((ANN, FC7), lambda: (0, 0)),
            pl.BlockSpec((ANN, ANN), lambda: (0, 0)),
            pl.BlockSpec((JEMB, FC7), lambda: (0, 0)),
            pl.BlockSpec((1, JEMB), lambda: (0, 0)),
        ],
        out_specs=[
            pl.BlockSpec((SENT, 1), lambda: (0, 0)),
            pl.BlockSpec((SENT, ANN), lambda: (0, 0)),
            pl.BlockSpec((SENT, JEMB), lambda: (0, 0)),
            pl.BlockSpec((SENT, 1), lambda: (0, 0)),
        ],
        out_shape=[
            jax.ShapeDtypeStruct((SENT, 1), jnp.int32),
            jax.ShapeDtypeStruct((SENT, ANN), jnp.float32),
            jax.ShapeDtypeStruct((SENT, JEMB), jnp.float32),
            jax.ShapeDtypeStruct((SENT, 1), jnp.float32),
        ],
        interpret=interpret,
    )(obj_attn, cxt_feats, dist_t, w1s, b2)

    sb = 8
    fuse = pl.pallas_call(
        _fuse_kernel,
        grid_spec=pltpu.PrefetchScalarGridSpec(
            num_scalar_prefetch=1,
            grid=(SENT // sb,),
            in_specs=[
                pl.BlockSpec((ANN, 5 * ANN), lambda i, ids: (0, 0)),
                pl.BlockSpec((sb, JEMB), lambda i, ids: (i, 0)),
                pl.BlockSpec((sb, ANN), lambda i, ids: (i, 0)),
                pl.BlockSpec((5, JEMB), lambda i, ids: (0, 0)),
            ],
            out_specs=pl.BlockSpec((sb, ANN, JEMB), lambda i, ids: (i, 0, 0)),
            scratch_shapes=[pltpu.VMEM((sb, 5 * ANN), jnp.float32)],
        ),
        out_shape=jax.ShapeDtypeStruct((SENT, ANN, JEMB), jnp.float32),
        interpret=interpret,
    )(maxid.reshape(SENT), cw, base, jnp.broadcast_to(okf, (SENT, ANN)), w2s)

    return fuse, dists, maxid[:, 0]


def kernel(cxt_feats, cxt_lfeats, obj_attn, wo_obj_idx, dist,
           fc7_norm_w, lfeat_norm_w, fc_w, fc_b):
    del wo_obj_idx
    return _run(cxt_feats, cxt_lfeats, obj_attn, dist, fc7_norm_w,
                lfeat_norm_w, fc_w, fc_b)


# speedup vs baseline: 7.1478x; 1.3086x over previous
import functools

import jax
import jax.numpy as jnp
from jax.experimental import pallas as pl
from jax.experimental.pallas import tpu as pltpu

SENT = 64
ANN = 256
FC7 = 2048
JEMB = 512

HI = jax.lax.Precision.HIGHEST


def _prologue_kernel(attn, cxt_feats, dist2, fc_w, w7, lw, b2,
                     maxid, dists, base, okf, w2s):
    a = attn[...]
    m = jnp.max(a, axis=1, keepdims=True)
    cols = jax.lax.broadcasted_iota(jnp.int32, (SENT, ANN), 1)
    ids = jnp.min(jnp.where(a == m, cols, ANN), axis=1,
                  keepdims=True)
    maxid[...] = ids
    onehot = (cols == ids).astype(jnp.float32)
    ok = jnp.where(m == 0.0, 0.0, 1.0)
    okf[...] = ok

    dg = jax.lax.dot_general(onehot, dist2[...],
                             (((1,), (1,)), ((), ())), precision=HI)
    dists[...] = jnp.where(ok == 0.0, 100.0, dg)

    w2s[...] = jnp.transpose(fc_w[:, FC7:FC7 + 5] * lw[...]) \
        .reshape(5, JEMB)

    rf = jax.lax.dot(onehot, cxt_feats[...], precision=HI)
    n = jnp.sqrt(jnp.sum(rf * rf, axis=1, keepdims=True))
    inv7 = ok / jnp.maximum(n, 1e-12)
    rfn = rf * inv7 * w7[...]
    base[...] = jax.lax.dot_general(rfn, fc_w[:, :FC7],
                                    (((1,), (1,)), ((), ()))) + b2[...]


def _fuse_kernel(ids_ref, cw, base, okf, w2s, fuse, scratch):
    sb = base.shape[0]
    pid = pl.program_id(0)

    for k in range(sb):
        idx = ids_ref[pid * sb + k]
        scratch[pl.ds(k, 1), :] = cw[pl.ds(idx, 1), :]
    g = scratch[...]

    lf = [g[:, c * ANN:(c + 1) * ANN] for c in range(5)]
    ss = lf[0] * lf[0]
    for c in range(1, 5):
        ss = ss + lf[c] * lf[c]
    invl = okf[...] / jnp.maximum(jnp.sqrt(ss), 1e-12)

    w2 = w2s[...]
    out = jnp.broadcast_to(base[...][:, None, :], (sb, ANN, JEMB))
    for c in range(5):
        out = out + (lf[c] * invl)[:, :, None] * w2[c][None, None, :]
    fuse[...] = out


@functools.partial(jax.jit, static_argnames=("interpret",))
def _run(cxt_feats, cxt_lfeats, obj_attn, dist, fc7_norm_w, lfeat_norm_w,
         fc_w, fc_b, interpret=False):
    cw = jnp.transpose(cxt_lfeats, (1, 2, 0)).reshape(ANN, 5 * ANN)
    dist2 = dist.reshape(ANN, ANN)
    b2 = fc_b.reshape(1, JEMB)

    maxid, dists, base, okf, w2s = pl.pallas_call(
        _prologue_kernel,
        in_specs=[
            pl.BlockSpec((SENT, ANN), lambda: (0, 0)),
            pl.BlockSpec((ANN, FC7), lambda: (0, 0)),
            pl.BlockSpec((ANN, ANN), lambda: (0, 0)),
            pl.BlockSpec((JEMB, FC7 + 5), lambda: (0, 0)),
            pl.BlockSpec((1, FC7), lambda: (0, 0)),
            pl.BlockSpec((1, 5), lambda: (0, 0)),
            pl.BlockSpec((1, JEMB), lambda: (0, 0)),
        ],
        out_specs=[
            pl.BlockSpec((SENT, 1), lambda: (0, 0)),
            pl.BlockSpec((SENT, ANN), lambda: (0, 0)),
            pl.BlockSpec((SENT, JEMB), lambda: (0, 0)),
            pl.BlockSpec((SENT, 1), lambda: (0, 0)),
            pl.BlockSpec((5, JEMB), lambda: (0, 0)),
        ],
        out_shape=[
            jax.ShapeDtypeStruct((SENT, 1), jnp.int32),
            jax.ShapeDtypeStruct((SENT, ANN), jnp.float32),
            jax.ShapeDtypeStruct((SENT, JEMB), jnp.float32),
            jax.ShapeDtypeStruct((SENT, 1), jnp.float32),
            jax.ShapeDtypeStruct((5, JEMB), jnp.float32),
        ],
        interpret=interpret,
    )(obj_attn, cxt_feats, dist2, fc_w, fc7_norm_w, lfeat_norm_w, b2)

    sb = 8
    fuse = pl.pallas_call(
        _fuse_kernel,
        grid_spec=pltpu.PrefetchScalarGridSpec(
            num_scalar_prefetch=1,
            grid=(SENT // sb,),
            in_specs=[
                pl.BlockSpec((ANN, 5 * ANN), lambda i, ids: (0, 0)),
                pl.BlockSpec((sb, JEMB), lambda i, ids: (i, 0)),
                pl.BlockSpec((sb, 1), lambda i, ids: (i, 0)),
                pl.BlockSpec((5, JEMB), lambda i, ids: (0, 0)),
            ],
            out_specs=pl.BlockSpec((sb, ANN, JEMB), lambda i, ids: (i, 0, 0)),
            scratch_shapes=[pltpu.VMEM((sb, 5 * ANN), jnp.float32)],
        ),
        out_shape=jax.ShapeDtypeStruct((SENT, ANN, JEMB), jnp.float32),
        interpret=interpret,
    )(maxid.reshape(SENT), cw, base, okf, w2s)

    return fuse, dists, maxid[:, 0]


def kernel(cxt_feats, cxt_lfeats, obj_attn, wo_obj_idx, dist,
           fc7_norm_w, lfeat_norm_w, fc_w, fc_b):
    del wo_obj_idx
    return _run(cxt_feats, cxt_lfeats, obj_attn, dist, fc7_norm_w,
                lfeat_norm_w, fc_w, fc_b)
